# speedup vs baseline: 6.0302x; 6.0302x over previous
import functools

import jax
import jax.numpy as jnp
from jax import lax
from jax.experimental import pallas as pl
from jax.experimental.pallas import tpu as pltpu
from jax.experimental.pallas import tpu_sc as plsc

_N = 50000
_D = 256
_B = 1024
_K = 128
_L = 32

_NC = 2
_NS = 16
_NW = _NC * _NS
_GPW = _B // _NW
_LANES = 16
_CHUNKS = _D // _LANES


def _pooled_sparsecore(table, idx):
    mesh = plsc.VectorSubcoreMesh(core_axis_name="c", subcore_axis_name="s")

    @functools.partial(
        pl.kernel,
        mesh=mesh,
        out_type=jax.ShapeDtypeStruct((_B, _D), jnp.float32),
        scratch_types=[
            pltpu.VMEM((_GPW, _K), jnp.int32),
            pltpu.VMEM((2, _K, _D), jnp.float32),
            pltpu.VMEM((_GPW, _D), jnp.float32),
            pltpu.SemaphoreType.DMA,
            pltpu.SemaphoreType.DMA,
        ],
    )
    def sc_kernel(table_hbm, idx_hbm, out_hbm, idx_v, rows_v, pooled_v,
                  sem0, sem1):
        sems = (sem0, sem1)
        wid = lax.axis_index("s") * _NC + lax.axis_index("c")
        base = wid * _GPW
        pltpu.sync_copy(idx_hbm.at[pl.ds(base, _GPW)], idx_v)

        pltpu.async_copy(table_hbm.at[idx_v.at[0]], rows_v.at[0], sems[0])

        for j in range(_GPW):
            cur = j % 2
            if j + 1 < _GPW:
                nxt = (j + 1) % 2
                pltpu.async_copy(table_hbm.at[idx_v.at[j + 1]],
                                 rows_v.at[nxt], sems[nxt])
            pltpu.make_async_copy(table_hbm.at[idx_v.at[j]],
                                  rows_v.at[cur], sems[cur]).wait()
            buf = rows_v.at[cur]

            def body(r, accs, buf=buf):
                return tuple(
                    accs[c] + buf[r, pl.ds(c * _LANES, _LANES)]
                    for c in range(_CHUNKS)
                )

            zeros = tuple(
                jnp.zeros((_LANES,), jnp.float32) for _ in range(_CHUNKS)
            )
            accs = lax.fori_loop(0, _K, body, zeros)
            for c in range(_CHUNKS):
                pooled_v[j, pl.ds(c * _LANES, _LANES)] = accs[c]

        pltpu.sync_copy(pooled_v, out_hbm.at[pl.ds(base, _GPW)])

    return sc_kernel(table, idx)


def _head_kernel(pooled_ref, w_ref, b_ref, out_ref):
    logits = (
        jnp.dot(pooled_ref[...], w_ref[...],
                preferred_element_type=jnp.float32)
        + b_ref[...][None, :]
    )
    m = jnp.max(logits, axis=1, keepdims=True)
    shifted = logits - m
    lse = jnp.log(jnp.sum(jnp.exp(shifted), axis=1, keepdims=True))
    out_ref[...] = shifted - lse


def _head(pooled, W, b):
    return pl.pallas_call(
        _head_kernel,
        out_shape=jax.ShapeDtypeStruct((_B, _L), jnp.float32),
    )(pooled, W, b)


def kernel(node_embedding_matrix, batch_x_index, W, b):
    pooled = _pooled_sparsecore(node_embedding_matrix, batch_x_index)
    return _head(pooled, W, b)


# speedup vs baseline: 6.5751x; 1.0904x over previous
import functools

import jax
import jax.numpy as jnp
from jax import lax
from jax.experimental import pallas as pl
from jax.experimental.pallas import tpu as pltpu
from jax.experimental.pallas import tpu_sc as plsc

_N = 50000
_D = 256
_B = 1024
_K = 128
_L = 32

_NC = 2
_NS = 16
_NW = _NC * _NS
_GPW = _B // _NW
_LANES = 16
_CHUNKS = _D // _LANES


def _pooled_sparsecore(table, idx):
    mesh = plsc.VectorSubcoreMesh(core_axis_name="c", subcore_axis_name="s")

    @functools.partial(
        pl.kernel,
        mesh=mesh,
        out_type=jax.ShapeDtypeStruct((_B, _D), jnp.float32),
        scratch_types=[
            pltpu.VMEM((_GPW, _K), jnp.int32),
            pltpu.VMEM((2, _K, _D), jnp.float32),
            pltpu.VMEM((_GPW, _D), jnp.float32),
            pltpu.SemaphoreType.DMA,
            pltpu.SemaphoreType.DMA,
        ],
    )
    def sc_kernel(table_hbm, idx_hbm, out_hbm, idx_v, rows_v, pooled_v,
                  sem0, sem1):
        sems = (sem0, sem1)
        wid = lax.axis_index("s") * _NC + lax.axis_index("c")
        base = wid * _GPW
        pltpu.sync_copy(idx_hbm.at[pl.ds(base, _GPW)], idx_v)

        def gather(j, slot):
            pltpu.async_copy(table_hbm.at[idx_v.at[j]], rows_v.at[slot],
                             sems[slot])

        def accumulate(j, slot):
            pltpu.make_async_copy(table_hbm.at[idx_v.at[j]],
                                  rows_v.at[slot], sems[slot]).wait()
            buf = rows_v.at[slot]

            def body(r, accs):
                r2 = 2 * r
                return tuple(
                    accs[c]
                    + buf[r2, pl.ds(c * _LANES, _LANES)]
                    + buf[r2 + 1, pl.ds(c * _LANES, _LANES)]
                    for c in range(_CHUNKS)
                )

            zeros = tuple(
                jnp.zeros((_LANES,), jnp.float32) for _ in range(_CHUNKS)
            )
            accs = lax.fori_loop(0, _K // 2, body, zeros)
            for c in range(_CHUNKS):
                pooled_v[j, pl.ds(c * _LANES, _LANES)] = accs[c]

        gather(0, 0)
        gather(1, 1)

        def outer(t, _):
            j0 = 2 * t
            accumulate(j0, 0)
            gather(j0 + 2, 0)
            accumulate(j0 + 1, 1)
            gather(j0 + 3, 1)
            return 0

        lax.fori_loop(0, _GPW // 2 - 1, outer, 0)
        accumulate(_GPW - 2, 0)
        accumulate(_GPW - 1, 1)

        pltpu.sync_copy(pooled_v, out_hbm.at[pl.ds(base, _GPW)])

    return sc_kernel(table, idx)


def _head_kernel(pooled_ref, w_ref, b_ref, out_ref):
    logits = (
        jnp.dot(pooled_ref[...], w_ref[...],
                preferred_element_type=jnp.float32)
        + b_ref[...][None, :]
    )
    m = jnp.max(logits, axis=1, keepdims=True)
    shifted = logits - m
    lse = jnp.log(jnp.sum(jnp.exp(shifted), axis=1, keepdims=True))
    out_ref[...] = shifted - lse


def _head(pooled, W, b):
    return pl.pallas_call(
        _head_kernel,
        out_shape=jax.ShapeDtypeStruct((_B, _L), jnp.float32),
    )(pooled, W, b)


def kernel(node_embedding_matrix, batch_x_index, W, b):
    pooled = _pooled_sparsecore(node_embedding_matrix, batch_x_index)
    return _head(pooled, W, b)


# speedup vs baseline: 7.2604x; 1.1042x over previous
import functools

import jax
import jax.numpy as jnp
from jax import lax
from jax.experimental import pallas as pl
from jax.experimental.pallas import tpu as pltpu
from jax.experimental.pallas import tpu_sc as plsc

_N = 50000
_D = 256
_B = 1024
_K = 128
_L = 32

_NC = 2
_NS = 16
_NW = _NC * _NS
_GPW = _B // _NW
_LANES = 16
_CHUNKS = _D // _LANES


def _pooled_sparsecore(table, idx):
    mesh = plsc.VectorSubcoreMesh(core_axis_name="c", subcore_axis_name="s")

    @functools.partial(
        pl.kernel,
        mesh=mesh,
        out_type=jax.ShapeDtypeStruct((_B, _D), jnp.float32),
        scratch_types=[
            pltpu.VMEM((_GPW, _K), jnp.int32),
            pltpu.VMEM((3, _K, _D), jnp.float32),
            pltpu.VMEM((_GPW, _D), jnp.float32),
            pltpu.SemaphoreType.DMA,
            pltpu.SemaphoreType.DMA,
            pltpu.SemaphoreType.DMA,
        ],
    )
    def sc_kernel(table_hbm, idx_hbm, out_hbm, idx_v, rows_v, pooled_v,
                  sem0, sem1, sem2):
        sems = (sem0, sem1, sem2)
        wid = lax.axis_index("s") * _NC + lax.axis_index("c")
        base = wid * _GPW
        pltpu.sync_copy(idx_hbm.at[pl.ds(base, _GPW)], idx_v)

        def gather(j, slot):
            pltpu.async_copy(table_hbm.at[idx_v.at[j]], rows_v.at[slot],
                             sems[slot])

        def accumulate(j, slot):
            pltpu.make_async_copy(table_hbm.at[idx_v.at[j]],
                                  rows_v.at[slot], sems[slot]).wait()
            buf = rows_v.at[slot]

            def body(r, accs):
                r2 = 2 * r
                return tuple(
                    accs[c]
                    + buf[r2, pl.ds(c * _LANES, _LANES)]
                    + buf[r2 + 1, pl.ds(c * _LANES, _LANES)]
                    for c in range(_CHUNKS)
                )

            zeros = tuple(
                jnp.zeros((_LANES,), jnp.float32) for _ in range(_CHUNKS)
            )
            accs = lax.fori_loop(0, _K // 2, body, zeros)
            for c in range(_CHUNKS):
                pooled_v[j, pl.ds(c * _LANES, _LANES)] = accs[c]

        gather(0, 0)
        gather(1, 1)
        gather(2, 2)

        def outer(t, _):
            j0 = 3 * t
            for u in range(3):
                accumulate(j0 + u, u)
                gather(j0 + u + 3, u)
            return 0

        lax.fori_loop(0, _GPW // 3 - 1, outer, 0)
        accumulate(27, 0)
        gather(30, 0)
        accumulate(28, 1)
        gather(31, 1)
        accumulate(29, 2)
        accumulate(30, 0)
        accumulate(31, 1)

        pltpu.sync_copy(pooled_v, out_hbm.at[pl.ds(base, _GPW)])

    return sc_kernel(table, idx)


def _head_kernel(pooled_ref, w_ref, b_ref, out_ref):
    logits = (
        jnp.dot(pooled_ref[...], w_ref[...],
                preferred_element_type=jnp.float32)
        + b_ref[...][None, :]
    )
    m = jnp.max(logits, axis=1, keepdims=True)
    shifted = logits - m
    lse = jnp.log(jnp.sum(jnp.exp(shifted), axis=1, keepdims=True))
    out_ref[...] = shifted - lse


def _head(pooled, W, b):
    return pl.pallas_call(
        _head_kernel,
        out_shape=jax.ShapeDtypeStruct((_B, _L), jnp.float32),
    )(pooled, W, b)


def kernel(node_embedding_matrix, batch_x_index, W, b):
    pooled = _pooled_sparsecore(node_embedding_matrix, batch_x_index)
    return _head(pooled, W, b)
